# Initial kernel scaffold; baseline (speedup 1.0000x reference)
import jax
import jax.numpy as jnp
from jax.experimental import pallas as pl


def kernel(x, edge_index, W1, b1, W2, b2):
    raise NotImplementedError("write your pallas kernel here")


# speedup vs baseline: 43.2899x; 43.2899x over previous
import functools

import jax
import jax.numpy as jnp
from jax import lax
from jax.experimental import pallas as pl
from jax.experimental.pallas import tpu as pltpu
from jax.experimental.pallas import tpu_sc as plsc

N = 10000
D_IN = 128
D_HID = 15
D_OUT = 128

NC = 2
NS = 16
NW = NC * NS
LANES = 16

NPAD = 10240
ROWS_PT = NPAD // NS
DUMMY = N

GROUP = 128
G_INNER = 8
CHUNK = GROUP * G_INNER

_MESH = plsc.VectorSubcoreMesh(
    core_axis_name="c", subcore_axis_name="s", num_cores=NC, num_subcores=NS)
_SC_PARAMS = pltpu.CompilerParams(use_tc_tiling_on_sc=False)


def _edge_chunks(e_total: int) -> tuple[int, int]:
    n_chunks = -(-e_total // CHUNK)
    return n_chunks, n_chunks * CHUNK


def _tile_span(wid, n_chunks):
    base, rem = n_chunks // NW, n_chunks % NW
    cnt = base + jnp.where(wid < rem, 1, 0)
    first = wid * base + jnp.minimum(wid, rem)
    return first, cnt


def _sc_degree(dst2d: jax.Array) -> jax.Array:
    n_chunks = dst2d.shape[0] // G_INNER

    @functools.partial(
        pl.kernel,
        out_type=jax.ShapeDtypeStruct((NC * NPAD, LANES), jnp.float32),
        mesh=_MESH,
        compiler_params=_SC_PARAMS,
        scratch_types=dict(
            acc=pltpu.VMEM_SHARED((NPAD, LANES), jnp.float32),
            didx=pltpu.VMEM((G_INNER, GROUP), jnp.int32),
            ones=pltpu.VMEM((GROUP, LANES), jnp.float32),
            zbuf=pltpu.VMEM((ROWS_PT, LANES), jnp.float32),
        ),
    )
    def kern(dst_hbm, out_hbm, acc, didx, ones, zbuf):
        cid = lax.axis_index("c")
        sid = lax.axis_index("s")
        wid = cid * NS + sid

        def fill(i, _):
            zbuf[i, :] = jnp.zeros((LANES,), jnp.float32)
            return 0
        lax.fori_loop(0, ROWS_PT, fill, 0)

        def fill1(i, _):
            ones[i, :] = jnp.full((LANES,), 1.0, jnp.float32)
            return 0
        lax.fori_loop(0, GROUP, fill1, 0)

        pltpu.sync_copy(zbuf, acc.at[pl.ds(sid * ROWS_PT, ROWS_PT)])
        plsc.subcore_barrier()

        first, cnt = _tile_span(wid, n_chunks)

        def step(t, _):
            g0 = (first + t) * G_INNER
            pltpu.sync_copy(dst_hbm.at[pl.ds(g0, G_INNER)], didx)
            for j in range(G_INNER):
                pltpu.sync_copy(ones, acc.at[didx.at[j]], add=True)
            return 0
        lax.fori_loop(0, cnt, step, 0)
        plsc.subcore_barrier()

        pltpu.sync_copy(acc.at[pl.ds(sid * ROWS_PT, ROWS_PT)], zbuf)
        pltpu.sync_copy(zbuf, out_hbm.at[pl.ds(cid * NPAD + sid * ROWS_PT, ROWS_PT)])

    return kern(dst2d)


def _sc_aggregate(src2d: jax.Array, dst2d: jax.Array, g: jax.Array) -> jax.Array:
    n_chunks = src2d.shape[0] // G_INNER

    @functools.partial(
        pl.kernel,
        out_type=jax.ShapeDtypeStruct((NC * NPAD, LANES), jnp.float32),
        mesh=_MESH,
        compiler_params=_SC_PARAMS,
        scratch_types=dict(
            acc=pltpu.VMEM_SHARED((NPAD, LANES), jnp.float32),
            sidx=pltpu.VMEM((G_INNER, GROUP), jnp.int32),
            didx=pltpu.VMEM((G_INNER, GROUP), jnp.int32),
            rows=pltpu.VMEM((G_INNER, GROUP, LANES), jnp.float32),
            zbuf=pltpu.VMEM((ROWS_PT, LANES), jnp.float32),
            sem=pltpu.SemaphoreType.DMA,
        ),
    )
    def kern(src_hbm, dst_hbm, g_hbm, out_hbm, acc, sidx, didx, rows, zbuf, sem):
        cid = lax.axis_index("c")
        sid = lax.axis_index("s")
        wid = cid * NS + sid

        def fill(i, _):
            zbuf[i, :] = jnp.zeros((LANES,), jnp.float32)
            return 0
        lax.fori_loop(0, ROWS_PT, fill, 0)
        pltpu.sync_copy(zbuf, acc.at[pl.ds(sid * ROWS_PT, ROWS_PT)])
        plsc.subcore_barrier()

        first, cnt = _tile_span(wid, n_chunks)

        def step(t, _):
            g0 = (first + t) * G_INNER
            pltpu.sync_copy(src_hbm.at[pl.ds(g0, G_INNER)], sidx)
            pltpu.sync_copy(dst_hbm.at[pl.ds(g0, G_INNER)], didx)
            copies = [
                pltpu.async_copy(g_hbm.at[sidx.at[j]], rows.at[j], sem)
                for j in range(G_INNER)
            ]
            for c in copies:
                c.wait()
            for j in range(G_INNER):
                pltpu.sync_copy(rows.at[j], acc.at[didx.at[j]], add=True)
            return 0
        lax.fori_loop(0, cnt, step, 0)
        plsc.subcore_barrier()

        pltpu.sync_copy(acc.at[pl.ds(sid * ROWS_PT, ROWS_PT)], zbuf)
        pltpu.sync_copy(zbuf, out_hbm.at[pl.ds(cid * NPAD + sid * ROWS_PT, ROWS_PT)])

    return kern(src2d, dst2d, g)


def _dis_from_parts(degp):
    deg = degp[:NPAD, :1] + degp[NPAD:, :1]
    return jnp.where(deg > 0, lax.rsqrt(jnp.maximum(deg, 1e-12)), 0.0)


def _tc_scale_matmul(x_p, w1p, degp):
    def body(x_ref, w_ref, d_ref, o_ref):
        dis = _dis_from_parts(d_ref[...])
        o_ref[...] = dis * jnp.dot(x_ref[...], w_ref[...],
                                   preferred_element_type=jnp.float32)
    return pl.pallas_call(
        body,
        out_shape=jax.ShapeDtypeStruct((NPAD, LANES), jnp.float32),
    )(x_p, w1p, degp)


def _tc_mid(aggp, degp, b1p):
    def body(a_ref, d_ref, b_ref, o_ref):
        dis = _dis_from_parts(d_ref[...])
        agg = a_ref[:NPAD, :] + a_ref[NPAD:, :]
        h = jnp.maximum(dis * agg + b_ref[...], 0.0)
        o_ref[...] = dis * h
    return pl.pallas_call(
        body,
        out_shape=jax.ShapeDtypeStruct((NPAD, LANES), jnp.float32),
    )(aggp, degp, b1p)


def _tc_out(aggp, degp, w2p, b2p):
    def body(a_ref, d_ref, w_ref, b_ref, o_ref):
        dis = _dis_from_parts(d_ref[...])
        agg = dis * (a_ref[:NPAD, :] + a_ref[NPAD:, :])
        o_ref[...] = jnp.dot(agg, w_ref[...],
                             preferred_element_type=jnp.float32) + b_ref[...]
    return pl.pallas_call(
        body,
        out_shape=jax.ShapeDtypeStruct((NPAD, D_OUT), jnp.float32),
    )(aggp, degp, w2p, b2p)


def kernel(x, edge_index, W1, b1, W2, b2):
    e_total = edge_index.shape[1] + N
    _, epad = _edge_chunks(e_total)
    pad = epad - e_total

    self_idx = jnp.arange(N, dtype=jnp.int32)
    src = jnp.concatenate(
        [edge_index[0].astype(jnp.int32), self_idx,
         jnp.zeros((pad,), jnp.int32)])
    dst = jnp.concatenate(
        [edge_index[1].astype(jnp.int32), self_idx,
         DUMMY + jnp.arange(pad, dtype=jnp.int32) % (NPAD - N)])
    src2d = src.reshape(-1, GROUP)
    dst2d = dst.reshape(-1, GROUP)

    x_p = jnp.zeros((NPAD, D_IN), jnp.float32).at[:N].set(x)
    w1p = jnp.zeros((D_IN, LANES), jnp.float32).at[:, :D_HID].set(W1)
    b1p = jnp.zeros((1, LANES), jnp.float32).at[0, :D_HID].set(b1)
    w2p = jnp.zeros((LANES, D_OUT), jnp.float32).at[:D_HID].set(W2)
    b2p = b2.reshape(1, D_OUT)

    degp = _sc_degree(dst2d)
    g1 = _tc_scale_matmul(x_p, w1p, degp)
    agg1 = _sc_aggregate(src2d, dst2d, g1)
    g2 = _tc_mid(agg1, degp, b1p)
    agg2 = _sc_aggregate(src2d, dst2d, g2)
    out = _tc_out(agg2, degp, w2p, b2p)
    return out[:N]
